# baseline (speedup 1.0000x reference)
import jax
import jax.numpy as jnp
from jax import lax
from jax.experimental import pallas as pl
from jax.experimental.pallas import tpu as pltpu

T = 8192
D = 2048
E = 64
TOPK = 2
TILE_T = 1024
NT = T // TILE_T


CHUNK = 128


def _routing_tail(acc, bias, scale, w_out_ref, i_out_ref, off):
    for c in range(0, acc.shape[1], CHUNK):
        _routing_tail_chunk(acc[:, c:c + CHUNK], bias, scale,
                            w_out_ref, i_out_ref, off + c)


def _routing_tail_chunk(acc, bias, scale, w_out_ref, i_out_ref, off):
    gate = acc[:E, :]
    up = acc[E:, :]
    s = jnp.abs(up * gate * jax.nn.sigmoid(gate))
    ex = jnp.exp(jnp.minimum(s, 80.0))
    sm = ex / jnp.sum(ex, axis=0, keepdims=True)

    biased = sm + bias
    row = jax.lax.broadcasted_iota(jnp.int32, biased.shape, 0)

    m1 = jnp.max(biased, axis=0, keepdims=True)
    i1 = jnp.min(jnp.where(biased == m1, row, E), axis=0, keepdims=True)
    mask1 = row == i1
    rest = jnp.where(mask1, -jnp.inf, biased)
    m2 = jnp.max(rest, axis=0, keepdims=True)
    i2 = jnp.min(jnp.where(rest == m2, row, E), axis=0, keepdims=True)
    mask2 = row == i2

    w = 1.0 + sm * scale
    w1 = jnp.sum(jnp.where(mask1, w, 0.0), axis=0, keepdims=True)
    w2 = jnp.sum(jnp.where(mask2, w, 0.0), axis=0, keepdims=True)

    w_out_ref[:, pl.ds(off, CHUNK)] = jnp.concatenate([w1, w2], axis=0)
    i_out_ref[:, pl.ds(off, CHUNK)] = jnp.concatenate([i1, i2], axis=0)


def _router_kernel(x_hbm, wg_ref, wu_ref, bias_ref, scale_ref,
                   w_out_ref, i_out_ref, xbuf_ref, wfull_ref, sems):
    wfull_ref[:, :E] = wg_ref[...]
    wfull_ref[:, E:] = wu_ref[...]
    bias = bias_ref[...]
    scale = scale_ref[...]
    dn = (((0,), (1,)), ((), ()))

    def copy(i, slot):
        return pltpu.make_async_copy(
            x_hbm.at[pl.ds(i * TILE_T, TILE_T), :],
            xbuf_ref.at[slot],
            sems.at[slot],
        )

    copy(0, 0).start()
    copy(1, 1).start()

    def step(k, carry):
        i0 = 2 * k
        copy(i0, 0).wait()
        acc = jax.lax.dot_general(
            wfull_ref[...], xbuf_ref[0],
            dimension_numbers=dn, preferred_element_type=jnp.float32)
        _routing_tail(acc, bias, scale, w_out_ref, i_out_ref, i0 * TILE_T)

        @pl.when(i0 + 2 < NT)
        def _pf0():
            copy(i0 + 2, 0).start()

        copy(i0 + 1, 1).wait()
        acc = jax.lax.dot_general(
            wfull_ref[...], xbuf_ref[1],
            dimension_numbers=dn, preferred_element_type=jnp.float32)
        _routing_tail(acc, bias, scale, w_out_ref, i_out_ref,
                      (i0 + 1) * TILE_T)

        @pl.when(i0 + 3 < NT)
        def _pf1():
            copy(i0 + 3, 1).start()

        return carry

    lax.fori_loop(0, NT // 2, step, 0)


@jax.jit
def kernel(x, W_gate, W_up, extra_scale, extra_bias):
    bias2d = extra_bias.reshape(E, 1)
    scale2d = extra_scale.reshape(E, 1)
    weights, indices = pl.pallas_call(
        _router_kernel,
        in_specs=[
            pl.BlockSpec(memory_space=pl.ANY),
            pl.BlockSpec(memory_space=pltpu.VMEM),
            pl.BlockSpec(memory_space=pltpu.VMEM),
            pl.BlockSpec(memory_space=pltpu.VMEM),
            pl.BlockSpec(memory_space=pltpu.VMEM),
        ],
        out_specs=[
            pl.BlockSpec(memory_space=pltpu.VMEM),
            pl.BlockSpec(memory_space=pltpu.VMEM),
        ],
        out_shape=[
            jax.ShapeDtypeStruct((TOPK, T), jnp.float32),
            jax.ShapeDtypeStruct((TOPK, T), jnp.int32),
        ],
        scratch_shapes=[
            pltpu.VMEM((2, TILE_T, D), jnp.float32),
            pltpu.VMEM((D, 2 * E), jnp.float32),
            pltpu.SemaphoreType.DMA((2,)),
        ],
    )(x, W_gate, W_up, bias2d, scale2d)
    return weights.T, indices.T


# speedup vs baseline: 1.0125x; 1.0125x over previous
import jax
import jax.numpy as jnp
from jax import lax
from jax.experimental import pallas as pl
from jax.experimental.pallas import tpu as pltpu

T = 8192
D = 2048
E = 64
TOPK = 2
TILE_T = 512
NT = T // TILE_T
NBUF = 4


CHUNK = 128


def _routing_tail(acc, bias, scale, w_out_ref, i_out_ref, off):
    for c in range(0, acc.shape[1], CHUNK):
        _routing_tail_chunk(acc[:, c:c + CHUNK], bias, scale,
                            w_out_ref, i_out_ref, off + c)


def _routing_tail_chunk(acc, bias, scale, w_out_ref, i_out_ref, off):
    gate = acc[:E, :]
    up = acc[E:, :]
    s = jnp.abs(up * gate * jax.nn.sigmoid(gate))
    ex = jnp.exp(jnp.minimum(s, 80.0))
    sm = ex / jnp.sum(ex, axis=0, keepdims=True)

    biased = sm + bias
    row = jax.lax.broadcasted_iota(jnp.int32, biased.shape, 0)

    m1 = jnp.max(biased, axis=0, keepdims=True)
    i1 = jnp.min(jnp.where(biased == m1, row, E), axis=0, keepdims=True)
    mask1 = row == i1
    rest = jnp.where(mask1, -jnp.inf, biased)
    m2 = jnp.max(rest, axis=0, keepdims=True)
    i2 = jnp.min(jnp.where(rest == m2, row, E), axis=0, keepdims=True)
    mask2 = row == i2

    w = 1.0 + sm * scale
    w1 = jnp.sum(jnp.where(mask1, w, 0.0), axis=0, keepdims=True)
    w2 = jnp.sum(jnp.where(mask2, w, 0.0), axis=0, keepdims=True)

    w_out_ref[:, pl.ds(off, CHUNK)] = jnp.concatenate([w1, w2], axis=0)
    i_out_ref[:, pl.ds(off, CHUNK)] = jnp.concatenate([i1, i2], axis=0)


def _router_kernel(x_hbm, wg_ref, wu_ref, bias_ref, scale_ref,
                   w_out_ref, i_out_ref, xbuf_ref, wfull_ref, sems):
    wfull_ref[:, :E] = wg_ref[...]
    wfull_ref[:, E:] = wu_ref[...]
    bias = bias_ref[...]
    scale = scale_ref[...]
    dn = (((0,), (1,)), ((), ()))

    def copy(i, slot):
        return pltpu.make_async_copy(
            x_hbm.at[pl.ds(i * TILE_T, TILE_T), :],
            xbuf_ref.at[slot],
            sems.at[slot],
        )

    for b in range(NBUF):
        copy(b, b).start()

    def step(k, carry):
        i0 = NBUF * k
        for b in range(NBUF):
            i = i0 + b
            copy(i, b).wait()
            acc = jax.lax.dot_general(
                wfull_ref[...], xbuf_ref[b],
                dimension_numbers=dn, preferred_element_type=jnp.float32)
            _routing_tail(acc, bias, scale, w_out_ref, i_out_ref,
                          i * TILE_T)

            @pl.when(i + NBUF < NT)
            def _pf():
                copy(i + NBUF, b).start()

        return carry

    lax.fori_loop(0, NT // NBUF, step, 0)


@jax.jit
def kernel(x, W_gate, W_up, extra_scale, extra_bias):
    bias2d = extra_bias.reshape(E, 1)
    scale2d = extra_scale.reshape(E, 1)
    weights, indices = pl.pallas_call(
        _router_kernel,
        in_specs=[
            pl.BlockSpec(memory_space=pl.ANY),
            pl.BlockSpec(memory_space=pltpu.VMEM),
            pl.BlockSpec(memory_space=pltpu.VMEM),
            pl.BlockSpec(memory_space=pltpu.VMEM),
            pl.BlockSpec(memory_space=pltpu.VMEM),
        ],
        out_specs=[
            pl.BlockSpec(memory_space=pltpu.VMEM),
            pl.BlockSpec(memory_space=pltpu.VMEM),
        ],
        out_shape=[
            jax.ShapeDtypeStruct((TOPK, T), jnp.float32),
            jax.ShapeDtypeStruct((TOPK, T), jnp.int32),
        ],
        scratch_shapes=[
            pltpu.VMEM((NBUF, TILE_T, D), jnp.float32),
            pltpu.VMEM((D, 2 * E), jnp.float32),
            pltpu.SemaphoreType.DMA((NBUF,)),
        ],
    )(x, W_gate, W_up, bias2d, scale2d)
    return weights.T, indices.T
